# Initial kernel scaffold; baseline (speedup 1.0000x reference)
import jax
import jax.numpy as jnp
from jax.experimental import pallas as pl


def kernel(x, edge_index, glove, W1, b1, W2, b2, W3, b3):
    raise NotImplementedError("write your pallas kernel here")


# speedup vs baseline: 20.8147x; 20.8147x over previous
import functools

import jax
import jax.numpy as jnp
from jax import lax
from jax.experimental import pallas as pl
from jax.experimental.pallas import tpu as pltpu
from jax.experimental.pallas import tpu_sc as plsc

_NC = 2
_NS = 16
_NW = _NC * _NS
_CHUNK = 128
_LANES = 16


def _fill(ref, rows, width, value):
    vec = jnp.full((_LANES,), value, jnp.float32)

    def row(i, _):
        for j in range(width // _LANES):
            ref[i, pl.ds(j * _LANES, _LANES)] = vec
        return 0

    lax.fori_loop(0, rows, row, 0)


@functools.lru_cache(maxsize=None)
def _make_sc_pass(width, n_pad, chunks, gather):
    rpt = n_pad // _NS
    zchunks = rpt // _CHUNK
    mesh = plsc.VectorSubcoreMesh(core_axis_name="c", subcore_axis_name="s")

    scratch = []
    if gather:
        scratch.append(pltpu.VMEM((chunks, _CHUNK), jnp.int32))
    scratch += [
        pltpu.VMEM((chunks, _CHUNK), jnp.int32),
        pltpu.VMEM((_CHUNK, width), jnp.float32),
        pltpu.VMEM((_CHUNK, width), jnp.float32),
        pltpu.VMEM_SHARED((n_pad, width), jnp.float32),
        pltpu.SemaphoreType.DMA,
    ]

    def body(*refs):
        if gather:
            (src3, dst3, table, out, srcv, dstv, rows, zbuf, acc, sem) = refs
        else:
            (dst3, out, dstv, rows, zbuf, acc, sem) = refs
        c = lax.axis_index("c")
        s = lax.axis_index("s")
        wid = c * _NS + s

        _fill(zbuf, _CHUNK, width, 0.0)
        if not gather:
            _fill(rows, _CHUNK, width, 1.0)

        def zinit(k, _):
            pltpu.sync_copy(zbuf, acc.at[pl.ds(s * rpt + k * _CHUNK, _CHUNK)])
            return 0

        lax.fori_loop(0, zchunks, zinit, 0)

        pltpu.sync_copy(dst3.at[wid], dstv)
        if gather:
            pltpu.sync_copy(src3.at[wid], srcv)
        plsc.subcore_barrier()

        def step(i, _):
            if gather:
                pltpu.async_copy(table.at[srcv.at[i]], rows, sem).wait()
            pltpu.sync_copy(rows, acc.at[dstv.at[i]], add=True)
            return 0

        lax.fori_loop(0, chunks, step, 0)
        plsc.subcore_barrier()

        pltpu.sync_copy(acc.at[pl.ds(s * rpt, rpt)],
                        out.at[c, pl.ds(s * rpt, rpt)])

    return pl.kernel(
        body,
        mesh=mesh,
        out_type=jax.ShapeDtypeStruct((_NC, n_pad, width), jnp.float32),
        scratch_types=scratch,
        compiler_params=pltpu.CompilerParams(use_tc_tiling_on_sc=False),
    )


def _tc_prep(x, glove, w1, d0, d1):
    n, _ = x.shape
    h = w1.shape[1]

    def body(x_ref, g_ref, w_ref, d0_ref, d1_ref, dinv_ref, hp_ref):
        deg = d0_ref[...] + d1_ref[...] + 1.0
        dinv = lax.rsqrt(jnp.maximum(deg, 1.0))
        dinv_ref[...] = dinv
        gw = jnp.dot(g_ref[...], w_ref[...],
                     preferred_element_type=jnp.float32)
        hm = jnp.dot(x_ref[...], gw, preferred_element_type=jnp.float32)
        hp_ref[...] = hm * dinv

    return pl.pallas_call(
        body,
        out_shape=[jax.ShapeDtypeStruct((n, 1), jnp.float32),
                   jax.ShapeDtypeStruct((n, h), jnp.float32)],
    )(x, glove, w1, d0, d1)


def _tc_mid(a0, a1, hp, dinv, b, wn):
    n = hp.shape[0]
    hn = wn.shape[1]

    def body(a0_ref, a1_ref, hp_ref, dinv_ref, b_ref, w_ref, out_ref):
        u = dinv_ref[...] * (a0_ref[...] + a1_ref[...] + hp_ref[...]) \
            + b_ref[...]
        v = jnp.maximum(u, 0.0)
        hm = jnp.dot(v, w_ref[...], preferred_element_type=jnp.float32)
        out_ref[...] = hm * dinv_ref[...]

    return pl.pallas_call(
        body,
        out_shape=jax.ShapeDtypeStruct((n, hn), jnp.float32),
    )(a0, a1, hp, dinv, b, wn)


def _tc_final(a0, a1, hp, dinv, b):
    n, c = hp.shape

    def body(a0_ref, a1_ref, hp_ref, dinv_ref, b_ref, out_ref):
        u = dinv_ref[...] * (a0_ref[...] + a1_ref[...] + hp_ref[...]) \
            + b_ref[...]
        m = jnp.max(u, axis=1, keepdims=True)
        su = u - m
        lse = jnp.log(jnp.sum(jnp.exp(su), axis=1, keepdims=True))
        out_ref[...] = su - lse

    return pl.pallas_call(
        body,
        out_shape=jax.ShapeDtypeStruct((n, c), jnp.float32),
    )(a0, a1, hp, dinv, b)


def kernel(x, edge_index, glove, W1, b1, W2, b2, W3, b3):
    n, _ = x.shape
    e = edge_index.shape[1]
    h = W1.shape[1]
    c = W3.shape[1]

    n_pad = -(-(n + 1) // (_NS * _CHUNK)) * (_NS * _CHUNK)
    chunks = -(-e // (_NW * _CHUNK))
    e_pad = _NW * chunks * _CHUNK

    src = edge_index[0].astype(jnp.int32)
    dst = edge_index[1].astype(jnp.int32)
    pad = e_pad - e
    src3 = jnp.concatenate(
        [src, jnp.zeros((pad,), jnp.int32)]).reshape(_NW, chunks, _CHUNK)
    dst3 = jnp.concatenate(
        [dst, jnp.full((pad,), n, jnp.int32)]).reshape(_NW, chunks, _CHUNK)

    c_pad = -(-c // _LANES) * _LANES
    w3p = jnp.pad(W3, ((0, 0), (0, c_pad - c)))

    deg_parts = _make_sc_pass(_LANES, n_pad, chunks, False)(dst3)
    d0 = deg_parts[0, :n, :1]
    d1 = deg_parts[1, :n, :1]

    dinv, hp1 = _tc_prep(x, glove, W1, d0, d1)

    layer_pass_h = _make_sc_pass(h, n_pad, chunks, True)
    s1 = layer_pass_h(src3, dst3, hp1)
    hp2 = _tc_mid(s1[0, :n], s1[1, :n], hp1, dinv, b1.reshape(1, h), W2)

    s2 = layer_pass_h(src3, dst3, hp2)
    hp3 = _tc_mid(s2[0, :n], s2[1, :n], hp2, dinv, b2.reshape(1, h), w3p)

    s3 = _make_sc_pass(c_pad, n_pad, chunks, True)(src3, dst3, hp3)
    return _tc_final(s3[0, :n, :c], s3[1, :n, :c], hp3[:, :c], dinv,
                     b3.reshape(1, c))


# speedup vs baseline: 20.9733x; 1.0076x over previous
import functools

import jax
import jax.numpy as jnp
from jax import lax
from jax.experimental import pallas as pl
from jax.experimental.pallas import tpu as pltpu
from jax.experimental.pallas import tpu_sc as plsc

_NC = 2
_NS = 16
_NW = _NC * _NS
_CHUNK = 128
_LANES = 16
_K = 8


def _fill(ref, rows, width, value):
    vec = jnp.full((_LANES,), value, jnp.float32)

    def row(i, _):
        for j in range(width // _LANES):
            ref[i, pl.ds(j * _LANES, _LANES)] = vec
        return 0

    lax.fori_loop(0, rows, row, 0)


@functools.lru_cache(maxsize=None)
def _make_sc_pass(width, n_pad, chunks, gather):
    rpt = n_pad // _NS
    zchunks = rpt // _CHUNK
    mesh = plsc.VectorSubcoreMesh(core_axis_name="c", subcore_axis_name="s")

    nbuf = _K if gather else 1
    scratch = []
    if gather:
        scratch.append(pltpu.VMEM((chunks, _CHUNK), jnp.int32))
    scratch += [
        pltpu.VMEM((chunks, _CHUNK), jnp.int32),
        [pltpu.VMEM((_CHUNK, width), jnp.float32)] * nbuf,
        pltpu.VMEM((_CHUNK, width), jnp.float32),
        pltpu.VMEM_SHARED((n_pad, width), jnp.float32),
        [pltpu.SemaphoreType.DMA] * _K,
        [pltpu.SemaphoreType.DMA] * _K,
    ]

    def body(*refs):
        if gather:
            (src3, dst3, table, out, srcv, dstv, rows, zbuf, acc,
             gsem, ssem) = refs
        else:
            (dst3, out, dstv, rows, zbuf, acc, gsem, ssem) = refs
        c = lax.axis_index("c")
        s = lax.axis_index("s")
        wid = c * _NS + s

        _fill(zbuf, _CHUNK, width, 0.0)
        if not gather:
            _fill(rows[0], _CHUNK, width, 1.0)

        def zinit(k, _):
            pltpu.sync_copy(zbuf, acc.at[pl.ds(s * rpt + k * _CHUNK, _CHUNK)])
            return 0

        lax.fori_loop(0, zchunks, zinit, 0)

        pltpu.sync_copy(dst3.at[wid], dstv)
        if gather:
            pltpu.sync_copy(src3.at[wid], srcv)
        plsc.subcore_barrier()

        def step(j, _):
            base = j * _K
            gh = []
            if gather:
                for g in range(_K):
                    gh.append(pltpu.async_copy(
                        table.at[srcv.at[base + g]], rows[g], gsem[g]))
            sh = []
            for g in range(_K):
                if gather:
                    gh[g].wait()
                sh.append(pltpu.async_copy(
                    rows[g if gather else 0], acc.at[dstv.at[base + g]],
                    ssem[g], add=True))
            for h in sh:
                h.wait()
            return 0

        lax.fori_loop(0, chunks // _K, step, 0)
        plsc.subcore_barrier()

        pltpu.sync_copy(acc.at[pl.ds(s * rpt, rpt)],
                        out.at[c, pl.ds(s * rpt, rpt)])

    return pl.kernel(
        body,
        mesh=mesh,
        out_type=jax.ShapeDtypeStruct((_NC, n_pad, width), jnp.float32),
        scratch_types=scratch,
        compiler_params=pltpu.CompilerParams(use_tc_tiling_on_sc=False),
    )


def _tc_prep(x, glove, w1, d0, d1):
    n, _ = x.shape
    h = w1.shape[1]

    def body(x_ref, g_ref, w_ref, d0_ref, d1_ref, dinv_ref, hp_ref):
        deg = d0_ref[...] + d1_ref[...] + 1.0
        dinv = lax.rsqrt(jnp.maximum(deg, 1.0))
        dinv_ref[...] = dinv
        gw = jnp.dot(g_ref[...], w_ref[...],
                     preferred_element_type=jnp.float32)
        hm = jnp.dot(x_ref[...], gw, preferred_element_type=jnp.float32)
        hp_ref[...] = hm * dinv

    return pl.pallas_call(
        body,
        out_shape=[jax.ShapeDtypeStruct((n, 1), jnp.float32),
                   jax.ShapeDtypeStruct((n, h), jnp.float32)],
    )(x, glove, w1, d0, d1)


def _tc_mid(a0, a1, hp, dinv, b, wn):
    n = hp.shape[0]
    hn = wn.shape[1]

    def body(a0_ref, a1_ref, hp_ref, dinv_ref, b_ref, w_ref, out_ref):
        u = dinv_ref[...] * (a0_ref[...] + a1_ref[...] + hp_ref[...]) \
            + b_ref[...]
        v = jnp.maximum(u, 0.0)
        hm = jnp.dot(v, w_ref[...], preferred_element_type=jnp.float32)
        out_ref[...] = hm * dinv_ref[...]

    return pl.pallas_call(
        body,
        out_shape=jax.ShapeDtypeStruct((n, hn), jnp.float32),
    )(a0, a1, hp, dinv, b, wn)


def _tc_final(a0, a1, hp, dinv, b):
    n, c = hp.shape

    def body(a0_ref, a1_ref, hp_ref, dinv_ref, b_ref, out_ref):
        u = dinv_ref[...] * (a0_ref[...] + a1_ref[...] + hp_ref[...]) \
            + b_ref[...]
        m = jnp.max(u, axis=1, keepdims=True)
        su = u - m
        lse = jnp.log(jnp.sum(jnp.exp(su), axis=1, keepdims=True))
        out_ref[...] = su - lse

    return pl.pallas_call(
        body,
        out_shape=jax.ShapeDtypeStruct((n, c), jnp.float32),
    )(a0, a1, hp, dinv, b)


def kernel(x, edge_index, glove, W1, b1, W2, b2, W3, b3):
    n, _ = x.shape
    e = edge_index.shape[1]
    h = W1.shape[1]
    c = W3.shape[1]

    n_pad = -(-(n + 1) // (_NS * _CHUNK)) * (_NS * _CHUNK)
    chunks = -(-e // (_NW * _CHUNK * _K)) * _K
    e_pad = _NW * chunks * _CHUNK

    src = edge_index[0].astype(jnp.int32)
    dst = edge_index[1].astype(jnp.int32)
    pad = e_pad - e
    src3 = jnp.concatenate(
        [src, jnp.zeros((pad,), jnp.int32)]).reshape(_NW, chunks, _CHUNK)
    dst3 = jnp.concatenate(
        [dst, jnp.full((pad,), n, jnp.int32)]).reshape(_NW, chunks, _CHUNK)

    c_pad = -(-c // _LANES) * _LANES
    w3p = jnp.pad(W3, ((0, 0), (0, c_pad - c)))

    deg_parts = _make_sc_pass(_LANES, n_pad, chunks, False)(dst3)
    d0 = deg_parts[0, :n, :1]
    d1 = deg_parts[1, :n, :1]

    dinv, hp1 = _tc_prep(x, glove, W1, d0, d1)

    layer_pass_h = _make_sc_pass(h, n_pad, chunks, True)
    s1 = layer_pass_h(src3, dst3, hp1)
    hp2 = _tc_mid(s1[0, :n], s1[1, :n], hp1, dinv, b1.reshape(1, h), W2)

    s2 = layer_pass_h(src3, dst3, hp2)
    hp3 = _tc_mid(s2[0, :n], s2[1, :n], hp2, dinv, b2.reshape(1, h), w3p)

    s3 = _make_sc_pass(c_pad, n_pad, chunks, True)(src3, dst3, hp3)
    return _tc_final(s3[0, :n, :c], s3[1, :n, :c], hp3[:, :c], dinv,
                     b3.reshape(1, c))


# speedup vs baseline: 33.8252x; 1.6128x over previous
import functools

import jax
import jax.numpy as jnp
from jax import lax
from jax.experimental import pallas as pl
from jax.experimental.pallas import tpu as pltpu
from jax.experimental.pallas import tpu_sc as plsc

_NC = 2
_NS = 16
_NW = _NC * _NS
_CHUNK = 128
_LANES = 16
_K = 8


def _fill(ref, rows, width, value):
    vec = jnp.full((_LANES,), value, jnp.float32)

    def row(i, _):
        for j in range(width // _LANES):
            ref[i, pl.ds(j * _LANES, _LANES)] = vec
        return 0

    lax.fori_loop(0, rows, row, 0)


@functools.lru_cache(maxsize=None)
def _make_sc_pass(width, n_pad, n_rows, chunks, gather):
    rpt = n_pad // _NS
    zchunks = rpt // _CHUNK
    trt = n_rows // _NS
    mesh = plsc.VectorSubcoreMesh(core_axis_name="c", subcore_axis_name="s")

    nbuf = _K if gather else 1
    scratch = []
    if gather:
        scratch.append(pltpu.VMEM((chunks, _CHUNK), jnp.int32))
    scratch += [
        pltpu.VMEM((chunks, _CHUNK), jnp.int32),
        [pltpu.VMEM((_CHUNK, width), jnp.float32)] * nbuf,
        pltpu.VMEM_SHARED((n_pad, width), jnp.float32),
        [pltpu.SemaphoreType.DMA] * _K,
        [pltpu.SemaphoreType.DMA] * _K,
    ]
    if gather:
        scratch.append(
            pltpu.VMEM_SHARED((n_rows, width), jnp.float32))

    def body(*refs):
        if gather:
            (src3, dst3, table, out, srcv, dstv, rows, acc,
             gsem, ssem, tab_s) = refs
        else:
            (dst3, out, dstv, rows, acc, gsem, ssem) = refs
        c = lax.axis_index("c")
        s = lax.axis_index("s")
        wid = c * _NS + s

        zbuf = rows[0]
        _fill(zbuf, _CHUNK, width, 0.0)

        def zinit(k, _):
            pltpu.sync_copy(zbuf, acc.at[pl.ds(s * rpt + k * _CHUNK, _CHUNK)])
            return 0

        lax.fori_loop(0, zchunks, zinit, 0)
        if not gather:
            _fill(rows[0], _CHUNK, width, 1.0)

        pltpu.sync_copy(dst3.at[wid], dstv)
        if gather:
            pltpu.sync_copy(src3.at[wid], srcv)
            pltpu.sync_copy(table.at[pl.ds(s * trt, trt)],
                            tab_s.at[pl.ds(s * trt, trt)])
        plsc.subcore_barrier()

        def step(j, _):
            base = j * _K
            gh = []
            if gather:
                for g in range(_K):
                    gh.append(pltpu.async_copy(
                        tab_s.at[srcv.at[base + g]], rows[g], gsem[g]))
            sh = []
            for g in range(_K):
                if gather:
                    gh[g].wait()
                sh.append(pltpu.async_copy(
                    rows[g if gather else 0], acc.at[dstv.at[base + g]],
                    ssem[g], add=True))
            for h in sh:
                h.wait()
            return 0

        lax.fori_loop(0, chunks // _K, step, 0)
        plsc.subcore_barrier()

        pltpu.sync_copy(acc.at[pl.ds(s * rpt, rpt)],
                        out.at[c, pl.ds(s * rpt, rpt)])

    return pl.kernel(
        body,
        mesh=mesh,
        out_type=jax.ShapeDtypeStruct((_NC, n_pad, width), jnp.float32),
        scratch_types=scratch,
        compiler_params=pltpu.CompilerParams(use_tc_tiling_on_sc=False),
    )


def _tc_prep(x, glove, w1, d0, d1):
    n, _ = x.shape
    h = w1.shape[1]

    def body(x_ref, g_ref, w_ref, d0_ref, d1_ref, dinv_ref, hp_ref):
        deg = d0_ref[...] + d1_ref[...] + 1.0
        dinv = lax.rsqrt(jnp.maximum(deg, 1.0))
        dinv_ref[...] = dinv
        gw = jnp.dot(g_ref[...], w_ref[...],
                     preferred_element_type=jnp.float32)
        hm = jnp.dot(x_ref[...], gw, preferred_element_type=jnp.float32)
        hp_ref[...] = hm * dinv

    return pl.pallas_call(
        body,
        out_shape=[jax.ShapeDtypeStruct((n, 1), jnp.float32),
                   jax.ShapeDtypeStruct((n, h), jnp.float32)],
    )(x, glove, w1, d0, d1)


def _tc_mid(a0, a1, hp, dinv, b, wn):
    n = hp.shape[0]
    hn = wn.shape[1]

    def body(a0_ref, a1_ref, hp_ref, dinv_ref, b_ref, w_ref, out_ref):
        u = dinv_ref[...] * (a0_ref[...] + a1_ref[...] + hp_ref[...]) \
            + b_ref[...]
        v = jnp.maximum(u, 0.0)
        hm = jnp.dot(v, w_ref[...], preferred_element_type=jnp.float32)
        out_ref[...] = hm * dinv_ref[...]

    return pl.pallas_call(
        body,
        out_shape=jax.ShapeDtypeStruct((n, hn), jnp.float32),
    )(a0, a1, hp, dinv, b, wn)


def _tc_final(a0, a1, hp, dinv, b):
    n, c = hp.shape

    def body(a0_ref, a1_ref, hp_ref, dinv_ref, b_ref, out_ref):
        u = dinv_ref[...] * (a0_ref[...] + a1_ref[...] + hp_ref[...]) \
            + b_ref[...]
        m = jnp.max(u, axis=1, keepdims=True)
        su = u - m
        lse = jnp.log(jnp.sum(jnp.exp(su), axis=1, keepdims=True))
        out_ref[...] = su - lse

    return pl.pallas_call(
        body,
        out_shape=jax.ShapeDtypeStruct((n, c), jnp.float32),
    )(a0, a1, hp, dinv, b)


def kernel(x, edge_index, glove, W1, b1, W2, b2, W3, b3):
    n, _ = x.shape
    e = edge_index.shape[1]
    h = W1.shape[1]
    c = W3.shape[1]

    n_pad = -(-(n + 1) // (_NS * _CHUNK)) * (_NS * _CHUNK)
    chunks = -(-e // (_NW * _CHUNK * _K)) * _K
    e_pad = _NW * chunks * _CHUNK

    src = edge_index[0].astype(jnp.int32)
    dst = edge_index[1].astype(jnp.int32)
    pad = e_pad - e
    src3 = jnp.concatenate(
        [src, jnp.zeros((pad,), jnp.int32)]).reshape(_NW, chunks, _CHUNK)
    dst3 = jnp.concatenate(
        [dst, jnp.full((pad,), n, jnp.int32)]).reshape(_NW, chunks, _CHUNK)

    c_pad = -(-c // _LANES) * _LANES
    w3p = jnp.pad(W3, ((0, 0), (0, c_pad - c)))
    n_tab = -(-n // _NS) * _NS

    deg_parts = _make_sc_pass(_LANES, n_pad, n_tab, chunks, False)(dst3)
    d0 = deg_parts[0, :n, :1]
    d1 = deg_parts[1, :n, :1]

    dinv, hp1 = _tc_prep(x, glove, W1, d0, d1)

    def padt(a):
        return jnp.pad(a, ((0, n_tab - n), (0, 0))) if n_tab != n else a

    layer_pass_h = _make_sc_pass(h, n_pad, n_tab, chunks, True)
    s1 = layer_pass_h(src3, dst3, padt(hp1))
    hp2 = _tc_mid(s1[0, :n], s1[1, :n], hp1, dinv, b1.reshape(1, h), W2)

    s2 = layer_pass_h(src3, dst3, padt(hp2))
    hp3 = _tc_mid(s2[0, :n], s2[1, :n], hp2, dinv, b2.reshape(1, h), w3p)

    s3 = _make_sc_pass(c_pad, n_pad, n_tab, chunks, True)(src3, dst3,
                                                          padt(hp3))
    return _tc_final(s3[0, :n, :c], s3[1, :n, :c], hp3[:, :c], dinv,
                     b3.reshape(1, c))


# speedup vs baseline: 37.8906x; 1.1202x over previous
import functools

import jax
import jax.numpy as jnp
from jax import lax
from jax.experimental import pallas as pl
from jax.experimental.pallas import tpu as pltpu
from jax.experimental.pallas import tpu_sc as plsc

_NC = 2
_NS = 16
_NW = _NC * _NS
_CHUNK = 128
_LANES = 16
_K = 8


def _fill(ref, rows, width, value):
    vec = jnp.full((_LANES,), value, jnp.float32)

    def row(i, _):
        for j in range(width // _LANES):
            ref[i, pl.ds(j * _LANES, _LANES)] = vec
        return 0

    lax.fori_loop(0, rows, row, 0)


@functools.lru_cache(maxsize=None)
def _make_sc_pass(width, n_pad, n_rows, chunks, gather):
    rpt = n_pad // _NS
    zchunks = rpt // _CHUNK
    trt = n_rows // _NS
    mesh = plsc.VectorSubcoreMesh(core_axis_name="c", subcore_axis_name="s")

    nbuf = _K if gather else 1
    scratch = []
    if gather:
        scratch.append(pltpu.VMEM((chunks, _CHUNK), jnp.int32))
    scratch += [
        pltpu.VMEM((chunks, _CHUNK), jnp.int32),
        [pltpu.VMEM((_CHUNK, width), jnp.float32)] * nbuf,
        pltpu.VMEM_SHARED((n_pad, width), jnp.float32),
        [pltpu.SemaphoreType.DMA] * _K,
        [pltpu.SemaphoreType.DMA] * _K,
    ]
    if gather:
        scratch.append(
            pltpu.VMEM_SHARED((n_rows, width), jnp.float32))

    def body(*refs):
        if gather:
            (src3, dst3, table, out, srcv, dstv, rows, acc,
             gsem, ssem, tab_s) = refs
        else:
            (dst3, out, dstv, rows, acc, gsem, ssem) = refs
        c = lax.axis_index("c")
        s = lax.axis_index("s")
        wid = c * _NS + s

        zbuf = rows[0]
        _fill(zbuf, _CHUNK, width, 0.0)

        def zinit(k, _):
            pltpu.sync_copy(zbuf, acc.at[pl.ds(s * rpt + k * _CHUNK, _CHUNK)])
            return 0

        stage = [pltpu.async_copy(dst3.at[wid], dstv, ssem[0])]
        if gather:
            stage.append(pltpu.async_copy(src3.at[wid], srcv, ssem[1]))
            stage.append(pltpu.async_copy(table.at[pl.ds(s * trt, trt)],
                                          tab_s.at[pl.ds(s * trt, trt)],
                                          ssem[2]))
        lax.fori_loop(0, zchunks, zinit, 0)
        if not gather:
            _fill(rows[0], _CHUNK, width, 1.0)
        for hc in stage:
            hc.wait()
        plsc.subcore_barrier()

        def step(j, _):
            base = j * _K
            gh = []
            if gather:
                for g in range(_K):
                    gh.append(pltpu.async_copy(
                        tab_s.at[srcv.at[base + g]], rows[g], gsem[g]))
            sh = []
            for g in range(_K):
                if gather:
                    gh[g].wait()
                sh.append(pltpu.async_copy(
                    rows[g if gather else 0], acc.at[dstv.at[base + g]],
                    ssem[g], add=True))
            for h in sh:
                h.wait()
            return 0

        lax.fori_loop(0, chunks // _K, step, 0)
        plsc.subcore_barrier()

        pltpu.sync_copy(acc.at[pl.ds(s * rpt, rpt)],
                        out.at[c, pl.ds(s * rpt, rpt)])

    return pl.kernel(
        body,
        mesh=mesh,
        out_type=jax.ShapeDtypeStruct((_NC, n_pad, width), jnp.float32),
        scratch_types=scratch,
        compiler_params=pltpu.CompilerParams(use_tc_tiling_on_sc=False),
    )


def _tc_prep(x, glove, w1, deg_parts):
    n, _ = x.shape
    h = w1.shape[1]

    def body(x_ref, g_ref, w_ref, dp_ref, dinv_ref, hp_ref):
        deg = dp_ref[0, :n, 0:1] + dp_ref[1, :n, 0:1] + 1.0
        dinv = lax.rsqrt(jnp.maximum(deg, 1.0))
        dinv_ref[...] = dinv
        gw = jnp.dot(g_ref[...], w_ref[...],
                     preferred_element_type=jnp.float32)
        hm = jnp.dot(x_ref[...], gw, preferred_element_type=jnp.float32)
        hp_ref[...] = hm * dinv

    return pl.pallas_call(
        body,
        out_shape=[jax.ShapeDtypeStruct((n, 1), jnp.float32),
                   jax.ShapeDtypeStruct((n, h), jnp.float32)],
    )(x, glove, w1, deg_parts)


def _tc_mid(sparts, hp, dinv, b, wn):
    n = hp.shape[0]
    hn = wn.shape[1]

    def body(s_ref, hp_ref, dinv_ref, b_ref, w_ref, out_ref):
        u = dinv_ref[...] * (s_ref[0, :n] + s_ref[1, :n] + hp_ref[...]) \
            + b_ref[...]
        v = jnp.maximum(u, 0.0)
        hm = jnp.dot(v, w_ref[...], preferred_element_type=jnp.float32)
        out_ref[...] = hm * dinv_ref[...]

    return pl.pallas_call(
        body,
        out_shape=jax.ShapeDtypeStruct((n, hn), jnp.float32),
    )(sparts, hp, dinv, b, wn)


def _tc_final(sparts, hp, dinv, b):
    n = hp.shape[0]
    c = b.shape[1]

    def body(s_ref, hp_ref, dinv_ref, b_ref, out_ref):
        u = dinv_ref[...] * (s_ref[0, :n, :c] + s_ref[1, :n, :c]
                             + hp_ref[:, :c]) + b_ref[...]
        m = jnp.max(u, axis=1, keepdims=True)
        su = u - m
        lse = jnp.log(jnp.sum(jnp.exp(su), axis=1, keepdims=True))
        out_ref[...] = su - lse

    return pl.pallas_call(
        body,
        out_shape=jax.ShapeDtypeStruct((n, c), jnp.float32),
    )(sparts, hp, dinv, b)


def kernel(x, edge_index, glove, W1, b1, W2, b2, W3, b3):
    n, _ = x.shape
    e = edge_index.shape[1]
    h = W1.shape[1]
    c = W3.shape[1]

    n_pad = -(-(n + 1) // (_NS * _CHUNK)) * (_NS * _CHUNK)
    chunks = -(-e // (_NW * _CHUNK * _K)) * _K
    e_pad = _NW * chunks * _CHUNK

    src = edge_index[0].astype(jnp.int32)
    dst = edge_index[1].astype(jnp.int32)
    pad = e_pad - e
    src3 = jnp.concatenate(
        [src, jnp.zeros((pad,), jnp.int32)]).reshape(_NW, chunks, _CHUNK)
    dst3 = jnp.concatenate(
        [dst, jnp.full((pad,), n, jnp.int32)]).reshape(_NW, chunks, _CHUNK)

    c_pad = -(-c // _LANES) * _LANES
    w3p = jnp.pad(W3, ((0, 0), (0, c_pad - c)))
    n_tab = -(-n // _NS) * _NS

    deg_parts = _make_sc_pass(_LANES, n_pad, n_tab, chunks, False)(dst3)
    dinv, hp1 = _tc_prep(x, glove, W1, deg_parts)

    def padt(a):
        return jnp.pad(a, ((0, n_tab - n), (0, 0))) if n_tab != n else a

    layer_pass_h = _make_sc_pass(h, n_pad, n_tab, chunks, True)
    s1 = layer_pass_h(src3, dst3, padt(hp1))
    hp2 = _tc_mid(s1, hp1, dinv, b1.reshape(1, h), W2)

    s2 = layer_pass_h(src3, dst3, padt(hp2))
    hp3 = _tc_mid(s2, hp2, dinv, b2.reshape(1, h), w3p)

    s3 = _make_sc_pass(c_pad, n_pad, n_tab, chunks, True)(src3, dst3,
                                                          padt(hp3))
    return _tc_final(s3, hp3, dinv, b3.reshape(1, c))
